# baseline (speedup 1.0000x reference)
import functools

import jax
import jax.numpy as jnp
from jax import lax
from jax.experimental import pallas as pl
from jax.experimental.pallas import tpu as pltpu
from jax.experimental.pallas import tpu_sc as plsc

_N, _C, _H, _W = 64, 768, 24, 24
_NW = 32
_ROWS_PER_W = _N // _NW
_SEG = 16
_SEGS = _C // _SEG
_NCHUNK = _ROWS_PER_W * _SEGS

_MESH = plsc.VectorSubcoreMesh(core_axis_name="c", subcore_axis_name="s")


@functools.partial(
    pl.kernel,
    mesh=_MESH,
    out_type=jax.ShapeDtypeStruct((_N, _C, _H, _W), jnp.float32),
    scratch_types=[
        pltpu.VMEM((_SEG, _H, _W), jnp.float32),
        pltpu.VMEM((_SEG, _H, _W), jnp.float32),
        pltpu.SemaphoreType.DMA,
        pltpu.SemaphoreType.DMA,
        pltpu.SemaphoreType.DMA,
        pltpu.SemaphoreType.DMA,
    ],
)
def _sc_copy(in_hbm, out_hbm, buf0, buf1, si0, si1, so0, so1):
    wid = lax.axis_index("s") * 2 + lax.axis_index("c")
    bufs = (buf0, buf1)
    isems = (si0, si1)
    osems = (so0, so1)

    def src(c):
        row = wid * _ROWS_PER_W + c // _SEGS
        return (row, pl.ds((c % _SEGS) * _SEG, _SEG))

    def in_copy(c, b):
        r, sl = src(c)
        return pltpu.make_async_copy(in_hbm.at[r, sl], bufs[b], isems[b])

    def out_copy(c, b):
        r, sl = src(c)
        return pltpu.make_async_copy(bufs[b], out_hbm.at[r, sl], osems[b])

    in_copy(0, 0).start()
    for c in range(_NCHUNK):
        b = c & 1
        if c + 1 < _NCHUNK:
            nb = (c + 1) & 1
            if c >= 1:
                out_copy(c - 1, nb).wait()
            in_copy(c + 1, nb).start()
        in_copy(c, b).wait()
        out_copy(c, b).start()
    out_copy(_NCHUNK - 2, _NCHUNK & 1).wait()
    out_copy(_NCHUNK - 1, (_NCHUNK - 1) & 1).wait()


def kernel(_input):
    return _sc_copy(_input)


# speedup vs baseline: 15.1965x; 15.1965x over previous
import jax
import jax.numpy as jnp
from jax.experimental import pallas as pl
from jax.experimental.pallas import tpu as pltpu


def _copy_body(in_ref, out_ref):
    out_ref[...] = in_ref[...]


def kernel(_input):
    n, c, h, w = _input.shape
    xt = jnp.transpose(_input, (0, 2, 3, 1))
    out = pl.pallas_call(
        _copy_body,
        grid=(n // 2,),
        in_specs=[pl.BlockSpec((2, h, w, c), lambda i: (i, 0, 0, 0))],
        out_specs=pl.BlockSpec((2, h, w, c), lambda i: (i, 0, 0, 0)),
        out_shape=jax.ShapeDtypeStruct((n, h, w, c), _input.dtype),
        compiler_params=pltpu.CompilerParams(
            dimension_semantics=("arbitrary",),
        ),
    )(xt)
    return jnp.transpose(out, (0, 3, 1, 2))
